# baseline (speedup 1.0000x reference)
import functools

import jax
import jax.numpy as jnp
from jax import lax
from jax.experimental import pallas as pl
from jax.experimental.pallas import tpu as pltpu
from jax.experimental.pallas import tpu_sc as plsc

NUM_CORES = 2
NUM_SUBCORES = 16
NUM_WORKERS = NUM_CORES * NUM_SUBCORES
CT = 21
CH = 256


def _sc_relayout(table):
    N, D = table.shape
    NM = D // 16
    n_tiles = N // 8
    tiles_per_w = n_tiles // NUM_WORKERS
    n_chunks = tiles_per_w // CT
    tail_tiles = n_tiles - NUM_WORKERS * tiles_per_w
    table3 = table.reshape(n_tiles, 8, D)
    mesh = plsc.VectorSubcoreMesh(
        core_axis_name="c", subcore_axis_name="s",
        num_cores=NUM_CORES, num_subcores=NUM_SUBCORES)

    @functools.partial(
        pl.kernel,
        mesh=mesh,
        out_type=jax.ShapeDtypeStruct((N, 128), jnp.float32),
        scratch_types=[
            pltpu.VMEM((CT, 8, D), jnp.float32),
            pltpu.VMEM((CT * 8, 128), jnp.float32),
            pltpu.SemaphoreType.DMA,
        ],
    )
    def relayout_kernel(table_hbm, out_hbm, vbuf, pbuf, sem):
        wid = lax.axis_index("s") * NUM_CORES + lax.axis_index("c")
        tbase = wid * tiles_per_w

        def do_chunk(t0, nt):
            pltpu.async_copy(table_hbm.at[pl.ds(t0, nt)],
                             vbuf.at[pl.ds(0, nt)], sem).wait()
            for k in range(nt):
                for s in range(8):
                    for m in range(NM):
                        pbuf[k * 8 + s, pl.ds(m * 16, 16)] = (
                            vbuf[k, s, pl.ds(m * 16, 16)])
            pltpu.sync_copy(pbuf.at[pl.ds(0, nt * 8)],
                            out_hbm.at[pl.ds(t0 * 8, nt * 8), :])

        def chunk_loop(c, carry):
            do_chunk(tbase + c * CT, CT)
            return carry

        lax.fori_loop(0, n_chunks, chunk_loop, 0)

        @pl.when(wid == NUM_WORKERS - 1)
        def _():
            do_chunk(NUM_WORKERS * tiles_per_w, tail_tiles)

    return relayout_kernel(table3)


def _sc_gather(table_pad, heads, tails, D):
    B = heads.shape[0]
    NM = D // 16
    b_per_w = B // NUM_WORKERS
    n_chunks = b_per_w // CH
    mesh = plsc.VectorSubcoreMesh(
        core_axis_name="c", subcore_axis_name="s",
        num_cores=NUM_CORES, num_subcores=NUM_SUBCORES)

    @functools.partial(
        pl.kernel,
        mesh=mesh,
        out_type=(
            jax.ShapeDtypeStruct((B, D), jnp.float32),
            jax.ShapeDtypeStruct((B, D), jnp.float32),
        ),
        scratch_types=[
            pltpu.VMEM((b_per_w,), jnp.int32),
            pltpu.VMEM((b_per_w,), jnp.int32),
            pltpu.VMEM((CH, 128), jnp.float32),
            pltpu.VMEM((CH, D), jnp.float32),
            pltpu.SemaphoreType.DMA,
        ],
    )
    def gather_kernel(table_hbm, heads_hbm, tails_hbm, outh_hbm, outt_hbm,
                      hidx_v, tidx_v, gbuf, obuf, sem):
        wid = lax.axis_index("s") * NUM_CORES + lax.axis_index("c")
        base = wid * b_per_w
        pltpu.sync_copy(heads_hbm.at[pl.ds(base, b_per_w)], hidx_v)
        pltpu.sync_copy(tails_hbm.at[pl.ds(base, b_per_w)], tidx_v)

        def one_table(idx_v, out_hbm):
            for c in range(n_chunks):
                pltpu.async_copy(
                    table_hbm.at[idx_v.at[pl.ds(c * CH, CH)]], gbuf, sem
                ).wait()
                for r in range(CH):
                    for m in range(NM):
                        obuf[r, pl.ds(m * 16, 16)] = gbuf[r, pl.ds(m * 16, 16)]
                pltpu.sync_copy(obuf, out_hbm.at[pl.ds(base + c * CH, CH)])

        one_table(hidx_v, outh_hbm)
        one_table(tidx_v, outt_hbm)

    return gather_kernel(table_pad, heads, tails)


def _tc_distance(head_rows, tail_rows, W, b):
    B, D = head_rows.shape
    BLK = 2048
    grid = (B // BLK,)

    def body(h_ref, t_ref, w_ref, b_ref, o_ref):
        y = jnp.dot(h_ref[...], w_ref[...].T,
                    preferred_element_type=jnp.float32)
        y = y + b_ref[...] - t_ref[...]
        d = jnp.sqrt(jnp.sum(y * y, axis=1))
        o_ref[...] = -d[None, :]

    out = pl.pallas_call(
        body,
        grid=grid,
        in_specs=[
            pl.BlockSpec((BLK, D), lambda i: (i, 0)),
            pl.BlockSpec((BLK, D), lambda i: (i, 0)),
            pl.BlockSpec((D, D), lambda i: (0, 0)),
            pl.BlockSpec((1, D), lambda i: (0, 0)),
        ],
        out_specs=pl.BlockSpec((1, BLK), lambda i: (0, i)),
        out_shape=jax.ShapeDtypeStruct((1, B), jnp.float32),
    )(head_rows, tail_rows, W, b.reshape(1, D))
    return out.reshape(B)


def kernel(heads, tails, entity_embeddings, W, b):
    D = entity_embeddings.shape[1]
    table_pad = _sc_relayout(entity_embeddings)
    head_rows, tail_rows = _sc_gather(table_pad, heads, tails, D)
    return _tc_distance(head_rows, tail_rows, W, b)


# speedup vs baseline: 1.5228x; 1.5228x over previous
import functools

import jax
import jax.numpy as jnp
from jax import lax
from jax.experimental import pallas as pl
from jax.experimental.pallas import tpu as pltpu
from jax.experimental.pallas import tpu_sc as plsc

NUM_CORES = 2
NUM_SUBCORES = 16
NUM_WORKERS = NUM_CORES * NUM_SUBCORES
CT = 21
CH = 256


def _sc_relayout(table):
    N, D = table.shape
    NM = D // 16
    n_tiles = N // 8
    tiles_per_w = n_tiles // NUM_WORKERS
    n_chunks = tiles_per_w // CT
    tail_tiles = n_tiles - NUM_WORKERS * tiles_per_w
    mesh = plsc.VectorSubcoreMesh(
        core_axis_name="c", subcore_axis_name="s",
        num_cores=NUM_CORES, num_subcores=NUM_SUBCORES)

    @functools.partial(
        pl.kernel,
        mesh=mesh,
        out_type=jax.ShapeDtypeStruct((N, 128), jnp.float32),
        scratch_types=[
            pltpu.VMEM((CT * 8, D), jnp.float32),
            pltpu.VMEM((CT * 8, 128), jnp.float32),
            pltpu.SemaphoreType.DMA,
        ],
    )
    def relayout_kernel(table_hbm, out_hbm, vbuf, pbuf, sem):
        wid = lax.axis_index("s") * NUM_CORES + lax.axis_index("c")
        tbase = wid * tiles_per_w

        def do_chunk(t0, nt):
            pltpu.async_copy(table_hbm.at[pl.ds(t0 * 8, nt * 8), :],
                             vbuf.at[pl.ds(0, nt * 8)], sem).wait()
            for r in range(nt * 8):
                for m in range(NM):
                    pbuf[r, pl.ds(m * 16, 16)] = vbuf[r, pl.ds(m * 16, 16)]
            pltpu.sync_copy(pbuf.at[pl.ds(0, nt * 8)],
                            out_hbm.at[pl.ds(t0 * 8, nt * 8), :])

        def chunk_loop(c, carry):
            do_chunk(tbase + c * CT, CT)
            return carry

        lax.fori_loop(0, n_chunks, chunk_loop, 0)

        @pl.when(wid == NUM_WORKERS - 1)
        def _():
            do_chunk(NUM_WORKERS * tiles_per_w, tail_tiles)

    return relayout_kernel(table)


def _sc_gather(table_pad, heads, tails, D):
    B = heads.shape[0]
    NM = D // 16
    b_per_w = B // NUM_WORKERS
    n_chunks = b_per_w // CH
    mesh = plsc.VectorSubcoreMesh(
        core_axis_name="c", subcore_axis_name="s",
        num_cores=NUM_CORES, num_subcores=NUM_SUBCORES)

    @functools.partial(
        pl.kernel,
        mesh=mesh,
        out_type=(
            jax.ShapeDtypeStruct((B, D), jnp.float32),
            jax.ShapeDtypeStruct((B, D), jnp.float32),
        ),
        scratch_types=[
            pltpu.VMEM((b_per_w,), jnp.int32),
            pltpu.VMEM((b_per_w,), jnp.int32),
            pltpu.VMEM((CH, 128), jnp.float32),
            pltpu.VMEM((CH, D), jnp.float32),
            pltpu.SemaphoreType.DMA,
        ],
    )
    def gather_kernel(table_hbm, heads_hbm, tails_hbm, outh_hbm, outt_hbm,
                      hidx_v, tidx_v, gbuf, obuf, sem):
        wid = lax.axis_index("s") * NUM_CORES + lax.axis_index("c")
        base = wid * b_per_w
        pltpu.sync_copy(heads_hbm.at[pl.ds(base, b_per_w)], hidx_v)
        pltpu.sync_copy(tails_hbm.at[pl.ds(base, b_per_w)], tidx_v)

        def one_table(idx_v, out_hbm):
            for c in range(n_chunks):
                pltpu.async_copy(
                    table_hbm.at[idx_v.at[pl.ds(c * CH, CH)]], gbuf, sem
                ).wait()
                for r in range(CH):
                    for m in range(NM):
                        obuf[r, pl.ds(m * 16, 16)] = gbuf[r, pl.ds(m * 16, 16)]
                pltpu.sync_copy(obuf, out_hbm.at[pl.ds(base + c * CH, CH)])

        one_table(hidx_v, outh_hbm)
        one_table(tidx_v, outt_hbm)

    return gather_kernel(table_pad, heads, tails)


def _tc_distance(head_rows, tail_rows, W, b):
    B, D = head_rows.shape
    BLK = 2048
    grid = (B // BLK,)

    def body(h_ref, t_ref, w_ref, b_ref, o_ref):
        y = jnp.dot(h_ref[...], w_ref[...].T,
                    preferred_element_type=jnp.float32)
        y = y + b_ref[...] - t_ref[...]
        d = jnp.sqrt(jnp.sum(y * y, axis=1))
        o_ref[...] = -d[None, :]

    out = pl.pallas_call(
        body,
        grid=grid,
        in_specs=[
            pl.BlockSpec((BLK, D), lambda i: (i, 0)),
            pl.BlockSpec((BLK, D), lambda i: (i, 0)),
            pl.BlockSpec((D, D), lambda i: (0, 0)),
            pl.BlockSpec((1, D), lambda i: (0, 0)),
        ],
        out_specs=pl.BlockSpec((1, BLK), lambda i: (0, i)),
        out_shape=jax.ShapeDtypeStruct((1, B), jnp.float32),
    )(head_rows, tail_rows, W, b.reshape(1, D))
    return out.reshape(B)


def kernel(heads, tails, entity_embeddings, W, b):
    D = entity_embeddings.shape[1]
    table_pad = _sc_relayout(entity_embeddings)
    head_rows, tail_rows = _sc_gather(table_pad, heads, tails, D)
    return _tc_distance(head_rows, tail_rows, W, b)
